# Initial kernel scaffold; baseline (speedup 1.0000x reference)
import jax
import jax.numpy as jnp
from jax.experimental import pallas as pl


def kernel(x, edge_index, W1, b1, W2, b2):
    raise NotImplementedError("write your pallas kernel here")


# speedup vs baseline: 42.7498x; 42.7498x over previous
import functools

import jax
import jax.numpy as jnp
from jax import lax
from jax.experimental import pallas as pl
from jax.experimental.pallas import tpu as pltpu
from jax.experimental.pallas import tpu_sc as plsc

N_NODES = 10000
D_IN = 256
D_HID = 16
D_OUT = 64

NC = 2
NS = 16
NW = NC * NS
PP = 10240
RPT = PP // NS
CR = PP * D_HID // 128
PACK8 = 128 // D_HID

_MESH = functools.partial(
    plsc.VectorSubcoreMesh,
    core_axis_name="c", subcore_axis_name="s",
    num_cores=NC, num_subcores=NS)

_SC_PARAMS = pltpu.CompilerParams(use_tc_tiling_on_sc=False)


def _zero_fill(buf, nrows):
    def body(i, _):
        buf[i, :] = jnp.zeros((D_HID,), jnp.float32)
        return 0
    lax.fori_loop(0, nrows, body, 0)


def _sc_propagate(h2d, src, dst, per_w):
    @functools.partial(
        pl.kernel,
        out_type=jax.ShapeDtypeStruct((NC * CR, 128), jnp.float32),
        mesh=_MESH(),
        compiler_params=_SC_PARAMS,
        scratch_types=[
            pltpu.VMEM((per_w,), jnp.int32),
            pltpu.VMEM((per_w,), jnp.int32),
            pltpu.VMEM((per_w, D_HID), jnp.float32),
            pltpu.VMEM((RPT, D_HID), jnp.float32),
            pltpu.VMEM((RPT * D_HID // 128, 128), jnp.float32),
            pltpu.VMEM_SHARED((PP, D_HID), jnp.float32),
            pltpu.SemaphoreType.DMA,
        ],
    )
    def prop_kernel(h_hbm, src_hbm, dst_hbm, out_hbm,
                    src_v, dst_v, rows_v, stage_v, pack_v, acc, sem):
        c = lax.axis_index("c")
        s = lax.axis_index("s")
        wid = s * NC + c
        row0 = s * RPT
        _zero_fill(stage_v, RPT)
        pltpu.sync_copy(stage_v, acc.at[pl.ds(row0, RPT)])
        pltpu.sync_copy(src_hbm.at[pl.ds(wid * per_w, per_w)], src_v)
        pltpu.sync_copy(dst_hbm.at[pl.ds(wid * per_w, per_w)], dst_v)
        plsc.subcore_barrier()
        pltpu.async_copy(h_hbm.at[src_v], rows_v, sem).wait()
        pltpu.sync_copy(rows_v, acc.at[dst_v], add=True)
        plsc.subcore_barrier()
        pltpu.sync_copy(acc.at[pl.ds(row0, RPT)], stage_v)
        def repack(i, _):
            v = stage_v[i, :]
            pack_v[i // PACK8, pl.ds((i % PACK8) * D_HID, D_HID)] = v
            return 0
        lax.fori_loop(0, RPT, repack, 0)
        prow = RPT * D_HID // 128
        pltpu.sync_copy(pack_v, out_hbm.at[pl.ds(c * CR + s * prow, prow)])

    return prop_kernel(h2d, src, dst)


PACK = 128 // D_HID
PR = N_NODES // PACK
TB = 320
TG = CR // TB
DO8 = PACK * D_OUT


def _pk(_):
    return pl.BlockSpec((TB, 128), lambda i: (i, 0))


def _q1(_):
    return pl.BlockSpec((TB, 128), lambda i: (i + TG, 0))


def _full(shape):
    return pl.BlockSpec(shape, lambda i: (0,) * len(shape))


def _mm_body(x8_ref, w1bd_ref, h_ref):
    h_ref[...] = jnp.dot(x8_ref[...], w1bd_ref[...],
                         preferred_element_type=jnp.float32)


def _tc_matmul1(x8, W1bd):
    return pl.pallas_call(
        _mm_body,
        grid=(TG,),
        in_specs=[
            pl.BlockSpec((TB, PACK * D_IN), lambda i: (i, 0)),
            _full((PACK * D_IN, 128)),
        ],
        out_specs=_pk(0),
        out_shape=jax.ShapeDtypeStruct((CR, 128), jnp.float32),
    )(x8, W1bd)


def _scale_body(hpre_ref, d0_ref, d1_ref, h1_ref, dis_ref):
    deg = d0_ref[...] + d1_ref[...] + 1.0
    dis = lax.rsqrt(deg)
    h1_ref[...] = hpre_ref[...] * dis
    dis_ref[...] = dis


def _tc_scale(hpre, deg):
    return pl.pallas_call(
        _scale_body,
        grid=(TG,),
        in_specs=[_pk(0), _pk(0), _q1(0)],
        out_specs=[_pk(0)] * 2,
        out_shape=[jax.ShapeDtypeStruct((CR, 128), jnp.float32)] * 2,
    )(hpre, deg, deg)


def _mid_body(p0_ref, p1_ref, h1_ref, dis_ref, b1_ref, h2_ref):
    dis = dis_ref[...]
    pre = dis * (p0_ref[...] + p1_ref[...] + h1_ref[...]) + b1_ref[...]
    h2_ref[...] = dis * jnp.maximum(pre, 0.0)


def _tc_mid(q1, h1p, dis, b1_128):
    return pl.pallas_call(
        _mid_body,
        grid=(TG,),
        in_specs=[_pk(0), _q1(0), _pk(0), _pk(0), _full((1, 128))],
        out_specs=_pk(0),
        out_shape=jax.ShapeDtypeStruct((CR, 128), jnp.float32),
    )(q1, q1, h1p, dis, b1_128)


def _z_body(p0_ref, p1_ref, h2_ref, dis_ref, z_ref):
    z_ref[...] = dis_ref[...] * (p0_ref[...] + p1_ref[...] + h2_ref[...])


def _tc_z(q2, h2p, dis):
    return pl.pallas_call(
        _z_body,
        grid=(TG,),
        in_specs=[_pk(0), _q1(0), _pk(0), _pk(0)],
        out_specs=_pk(0),
        out_shape=jax.ShapeDtypeStruct((CR, 128), jnp.float32),
    )(q2, q2, h2p, dis)


def _out_body(z_ref, w2bd_ref, b2_ref, gsum_ref, o_ref):
    o = jnp.dot(z_ref[...], w2bd_ref[...], preferred_element_type=jnp.float32)
    o = o + b2_ref[...]
    m = jnp.max(o, axis=1, keepdims=True)
    ex = jnp.exp(o - m)
    s = jnp.dot(ex, gsum_ref[...], preferred_element_type=jnp.float32)
    o_ref[...] = o - m - jnp.log(s)


def _tc_out(z, W2bd, b2_512, gsum):
    return pl.pallas_call(
        _out_body,
        grid=(TG,),
        in_specs=[_pk(0),
                  _full((128, DO8)), _full((1, DO8)), _full((DO8, DO8))],
        out_specs=pl.BlockSpec((TB, DO8), lambda i: (i, 0)),
        out_shape=jax.ShapeDtypeStruct((PR, DO8), jnp.float32),
    )(z, W2bd, b2_512, gsum)


def kernel(x, edge_index, W1, b1, W2, b2):
    n, e = x.shape[0], edge_index.shape[1]
    assert n == N_NODES and e % NW == 0 and (e // NW) % 8 == 0

    src_1 = edge_index[0].astype(jnp.int32).reshape(e)
    dst_1 = edge_index[1].astype(jnp.int32).reshape(e)

    x8 = x.reshape(PR, PACK * D_IN)
    eye8 = jnp.eye(PACK, dtype=jnp.float32)
    W1bd = jnp.kron(eye8, W1)
    W2bd = jnp.kron(eye8, W2)
    gsum = jnp.kron(eye8, jnp.ones((D_OUT, D_OUT), jnp.float32))
    b1_128 = jnp.tile(b1, PACK).reshape(1, 128)
    b2_512 = jnp.tile(b2, PACK).reshape(1, DO8)

    ones_2 = jnp.ones((PP, D_HID), jnp.float32)
    deg = _sc_propagate(ones_2, src_1, dst_1, e // NW)
    hpre = _tc_matmul1(x8, W1bd)
    h1p, dis = _tc_scale(hpre, deg)
    q1 = _sc_propagate(h1p.reshape(PP, D_HID), src_1, dst_1, e // NW)
    h2p = _tc_mid(q1, h1p, dis, b1_128)
    q2 = _sc_propagate(h2p.reshape(PP, D_HID), src_1, dst_1, e // NW)
    z = _tc_z(q2, h2p, dis)
    out = _tc_out(z, W2bd, b2_512, gsum)
    return out.reshape(n, D_OUT)


# speedup vs baseline: 49.2293x; 1.1516x over previous
import functools

import jax
import jax.numpy as jnp
from jax import lax
from jax.experimental import pallas as pl
from jax.experimental.pallas import tpu as pltpu
from jax.experimental.pallas import tpu_sc as plsc

N_NODES = 10000
D_IN = 256
D_HID = 16
D_OUT = 64

NC = 2
NS = 16
NW = NC * NS
PP = 10240
RPT = PP // NS
CR = PP * D_HID // 128
PACK8 = 128 // D_HID

_MESH = functools.partial(
    plsc.VectorSubcoreMesh,
    core_axis_name="c", subcore_axis_name="s",
    num_cores=NC, num_subcores=NS)

_SC_PARAMS = pltpu.CompilerParams(use_tc_tiling_on_sc=False)


def _sc_propagate(h2d, zeros2d, ei, per_w):
    @functools.partial(
        pl.kernel,
        out_type=jax.ShapeDtypeStruct((NC * CR, 128), jnp.float32),
        mesh=_MESH(),
        compiler_params=_SC_PARAMS,
        scratch_types=[
            pltpu.VMEM((per_w,), jnp.int32),
            pltpu.VMEM((per_w,), jnp.int32),
            pltpu.VMEM((per_w, D_HID), jnp.float32),
            pltpu.VMEM((RPT, D_HID), jnp.float32),
            pltpu.VMEM((RPT * D_HID // 128, 128), jnp.float32),
            pltpu.VMEM_SHARED((PP, D_HID), jnp.float32),
            pltpu.SemaphoreType.DMA,
        ],
    )
    def prop_kernel(h_hbm, z_hbm, ei_hbm, out_hbm,
                    src_v, dst_v, rows_v, stage_v, pack_v, acc, sem):
        c = lax.axis_index("c")
        s = lax.axis_index("s")
        wid = s * NC + c
        row0 = s * RPT
        pltpu.sync_copy(z_hbm.at[pl.ds(row0, RPT)], acc.at[pl.ds(row0, RPT)])
        pltpu.sync_copy(ei_hbm.at[pl.ds(wid * per_w, per_w)], src_v)
        pltpu.sync_copy(ei_hbm.at[pl.ds(NW * per_w + wid * per_w, per_w)],
                        dst_v)
        plsc.subcore_barrier()
        pltpu.async_copy(h_hbm.at[src_v], rows_v, sem).wait()
        pltpu.sync_copy(rows_v, acc.at[dst_v], add=True)
        plsc.subcore_barrier()
        pltpu.sync_copy(acc.at[pl.ds(row0, RPT)], stage_v)
        def repack(i, _):
            v = stage_v[i, :]
            pack_v[i // PACK8, pl.ds((i % PACK8) * D_HID, D_HID)] = v
            return 0
        lax.fori_loop(0, RPT, repack, 0)
        prow = RPT * D_HID // 128
        pltpu.sync_copy(pack_v, out_hbm.at[pl.ds(c * CR + s * prow, prow)])

    return prop_kernel(h2d, zeros2d, ei)


PACK = 128 // D_HID
PR = N_NODES // PACK
TB = 320
TG = CR // TB
DO8 = PACK * D_OUT


def _pk(_):
    return pl.BlockSpec((TB, 128), lambda i: (i, 0))


def _q1(_):
    return pl.BlockSpec((TB, 128), lambda i: (i + TG, 0))


def _full(shape):
    return pl.BlockSpec(shape, lambda i: (0,) * len(shape))


def _eprep_body(ei_ref, o_ref):
    e = ei_ref.shape[1]
    o_ref[pl.ds(0, e)] = ei_ref[0, :]
    o_ref[pl.ds(e, e)] = ei_ref[1, :]


def _tc_eprep(edge_index):
    e = edge_index.shape[1]
    return pl.pallas_call(
        _eprep_body,
        out_shape=jax.ShapeDtypeStruct((2 * e,), jnp.int32),
    )(edge_index)


def _mm_body(x8_ref, w1bd_ref, h_ref):
    h_ref[...] = jnp.dot(x8_ref[...], w1bd_ref[...],
                         preferred_element_type=jnp.float32)


def _tc_matmul1(x8, W1bd):
    return pl.pallas_call(
        _mm_body,
        grid=(TG,),
        in_specs=[
            pl.BlockSpec((TB, PACK * D_IN), lambda i: (i, 0)),
            _full((PACK * D_IN, 128)),
        ],
        out_specs=_pk(0),
        out_shape=jax.ShapeDtypeStruct((CR, 128), jnp.float32),
    )(x8, W1bd)


def _scale_body(hpre_ref, d0_ref, d1_ref, h1_ref, dis_ref):
    deg = d0_ref[...] + d1_ref[...] + 1.0
    dis = lax.rsqrt(deg)
    h1_ref[...] = hpre_ref[...] * dis
    dis_ref[...] = dis


def _tc_scale(hpre, deg):
    return pl.pallas_call(
        _scale_body,
        grid=(TG,),
        in_specs=[_pk(0), _pk(0), _q1(0)],
        out_specs=[_pk(0)] * 2,
        out_shape=[jax.ShapeDtypeStruct((CR, 128), jnp.float32)] * 2,
    )(hpre, deg, deg)


def _mid_body(p0_ref, p1_ref, h1_ref, dis_ref, b1_ref, h2_ref):
    dis = dis_ref[...]
    pre = dis * (p0_ref[...] + p1_ref[...] + h1_ref[...]) + b1_ref[...]
    h2_ref[...] = dis * jnp.maximum(pre, 0.0)


def _tc_mid(q1, h1p, dis, b1_128):
    return pl.pallas_call(
        _mid_body,
        grid=(TG,),
        in_specs=[_pk(0), _q1(0), _pk(0), _pk(0), _full((1, 128))],
        out_specs=_pk(0),
        out_shape=jax.ShapeDtypeStruct((CR, 128), jnp.float32),
    )(q1, q1, h1p, dis, b1_128)


def _out_body(p0_ref, p1_ref, h2_ref, dis_ref, w2bd_ref, b2_ref, gsum_ref,
              o_ref):
    z = dis_ref[...] * (p0_ref[...] + p1_ref[...] + h2_ref[...])
    o = jnp.dot(z, w2bd_ref[...], preferred_element_type=jnp.float32)
    o = o + b2_ref[...]
    m = jnp.max(o, axis=1, keepdims=True)
    ex = jnp.exp(o - m)
    s = jnp.dot(ex, gsum_ref[...], preferred_element_type=jnp.float32)
    o_ref[...] = o - m - jnp.log(s)


def _tc_out(q2, h2p, dis, W2bd, b2_512, gsum):
    return pl.pallas_call(
        _out_body,
        grid=(TG,),
        in_specs=[_pk(0), _q1(0), _pk(0), _pk(0),
                  _full((128, DO8)), _full((1, DO8)), _full((DO8, DO8))],
        out_specs=pl.BlockSpec((TB, DO8), lambda i: (i, 0)),
        out_shape=jax.ShapeDtypeStruct((PR, DO8), jnp.float32),
    )(q2, q2, h2p, dis, W2bd, b2_512, gsum)


def kernel(x, edge_index, W1, b1, W2, b2):
    n, e = x.shape[0], edge_index.shape[1]
    assert n == N_NODES and e % NW == 0 and (e // NW) % 8 == 0
    per_w = e // NW

    x8 = x.reshape(PR, PACK * D_IN)
    eye8 = jnp.eye(PACK, dtype=jnp.float32)
    W1bd = jnp.kron(eye8, W1)
    W2bd = jnp.kron(eye8, W2)
    gsum = jnp.kron(eye8, jnp.ones((D_OUT, D_OUT), jnp.float32))
    b1_128 = jnp.tile(b1, PACK).reshape(1, 128)
    b2_512 = jnp.tile(b2, PACK).reshape(1, DO8)
    zeros2d = jnp.zeros((PP, D_HID), jnp.float32)
    ones2d = jnp.ones((PP, D_HID), jnp.float32)

    ei = _tc_eprep(edge_index.astype(jnp.int32))
    deg = _sc_propagate(ones2d, zeros2d, ei, per_w)
    hpre = _tc_matmul1(x8, W1bd)
    h1p, dis = _tc_scale(hpre, deg)
    q1 = _sc_propagate(h1p.reshape(PP, D_HID), zeros2d, ei, per_w)
    h2p = _tc_mid(q1, h1p, dis, b1_128)
    q2 = _sc_propagate(h2p.reshape(PP, D_HID), zeros2d, ei, per_w)
    out = _tc_out(q2, h2p, dis, W2bd, b2_512, gsum)
    return out.reshape(n, D_OUT)
